# Initial kernel scaffold; baseline (speedup 1.0000x reference)
import jax
import jax.numpy as jnp
from jax.experimental import pallas as pl


def kernel(x, Wq, bq, Wk, bk, Wv, bv, Wo, bo, index_sample):
    raise NotImplementedError("write your pallas kernel here")


# speedup vs baseline: 3.3776x; 3.3776x over previous
import functools
import math

import jax
import jax.numpy as jnp
from jax.experimental import pallas as pl
from jax.experimental.pallas import tpu as pltpu

N_HEADS = 12
FACTOR = 5
NEG_INF = float("-inf")


def _qkv_body(x_ref, wq_ref, wk_ref, wv_ref, bq_ref, bk_ref, bv_ref,
              q_ref, k_ref, v_ref):
    xb = x_ref[0]
    cn = (((1,), (0,)), ((), ()))
    f32 = jnp.float32
    q_ref[0, 0] = jax.lax.dot_general(xb, wq_ref[0], cn,
                                      preferred_element_type=f32) + bq_ref[0]
    k_ref[0, 0] = jax.lax.dot_general(xb, wk_ref[0], cn,
                                      preferred_element_type=f32) + bk_ref[0]
    v_ref[0, 0] = jax.lax.dot_general(xb, wv_ref[0], cn,
                                      preferred_element_type=f32) + bv_ref[0]


def _wcount_body(idx_ref, w_ref):
    blk, u_part = idx_ref.shape
    l_keys = w_ref.shape[1]
    idx_f = idx_ref[...].astype(jnp.float32)
    k_iota = jax.lax.broadcasted_iota(jnp.int32, (blk, l_keys), 1).astype(jnp.float32)
    s_iota = jax.lax.broadcasted_iota(jnp.int32, (u_part, 1), 0)

    def body(s, acc):
        e = (s_iota == s).astype(jnp.float32)
        col = jax.lax.dot_general(idx_f, e, (((1,), (0,)), ((), ())),
                                  precision=jax.lax.Precision.HIGHEST,
                                  preferred_element_type=jnp.float32)
        return acc + (k_iota == col).astype(jnp.float32)

    acc = jax.lax.fori_loop(0, u_part, body, jnp.zeros((blk, l_keys), jnp.float32))
    w_ref[...] = acc.astype(jnp.int8)


def _attn_body(q_ref, k_ref, v_ref, w_ref, ctx_ref, m_ref, *, u, blk):
    L, dh = q_ref.shape[2], q_ref.shape[3]
    nblk = L // blk
    upad = 64
    cn_t = (((1,), (1,)), ((), ()))
    cn_m = (((1,), (0,)), ((), ()))
    f32 = jnp.float32

    kv = k_ref[0, 0]
    vv = v_ref[0, 0]

    def m_block(i, _):
        qb = q_ref[0, 0, pl.ds(i * blk, blk), :]
        s = jax.lax.dot_general(qb, kv, cn_t, preferred_element_type=f32)
        cnt = w_ref[pl.ds(i * blk, blk), :].astype(f32)
        smax = jnp.max(jnp.where(cnt > 0, s, NEG_INF), axis=1)
        ssum = jnp.sum(s * cnt, axis=1)
        mb = smax - ssum * (1.0 / L)
        m_ref[pl.ds(i, 1), :] = mb.reshape(1, blk)
        return 0

    jax.lax.fori_loop(0, nblk, m_block, 0)
    m_all = m_ref[...]

    pos = (jax.lax.broadcasted_iota(jnp.int32, (nblk, blk), 0) * blk
           + jax.lax.broadcasted_iota(jnp.int32, (nblk, blk), 1))
    lane = jax.lax.broadcasted_iota(jnp.int32, (1, upad), 1)

    def topk_body(j, carry):
        m_cur, tops = carry
        mmax = jnp.max(m_cur)
        sel = jnp.min(jnp.where(m_cur == mmax, pos, 2 * L))
        tops = jnp.where(lane == j, sel, tops)
        m_cur = jnp.where(pos == sel, NEG_INF, m_cur)
        return m_cur, tops

    tops0 = jnp.full((1, upad), 2 * L, jnp.int32)
    _, tops = jax.lax.fori_loop(0, u, topk_body, (m_all, tops0))
    tops_col = tops.reshape(upad, 1)

    k_iota = jax.lax.broadcasted_iota(jnp.int32, (upad, L), 1)
    onehot = (k_iota == tops_col).astype(f32)
    q_sel = jax.lax.dot_general(onehot, q_ref[0, 0], cn_m,
                                precision=jax.lax.Precision.HIGHEST, preferred_element_type=f32)
    scores = jax.lax.dot_general(q_sel, kv, cn_t, preferred_element_type=f32)
    scores = scores * (1.0 / math.sqrt(dh))
    scores = jnp.where(k_iota > tops_col, NEG_INF, scores)
    scores = scores - jnp.max(scores, axis=1, keepdims=True)
    p = jnp.exp(scores)
    attn = p / jnp.sum(p, axis=1, keepdims=True)
    update = jax.lax.dot_general(attn, vv, cn_m,
                                 preferred_element_type=f32)

    tri = (jax.lax.broadcasted_iota(jnp.int32, (blk, blk), 0)
           >= jax.lax.broadcasted_iota(jnp.int32, (blk, blk), 1)).astype(f32)
    row_iota = jax.lax.broadcasted_iota(jnp.int32, (blk, 1), 0)

    def ctx_block(i, carry):
        vb = v_ref[0, 0, pl.ds(i * blk, blk), :]
        csum = jax.lax.dot_general(tri, vb, cn_m,
                                   precision=jax.lax.Precision.HIGHEST, preferred_element_type=f32) + carry
        offs = row_iota + i * blk
        selmat = (offs == tops).astype(f32)
        repl = jax.lax.dot_general(selmat, update, cn_m,
                                   precision=jax.lax.Precision.HIGHEST, preferred_element_type=f32)
        issel = jnp.sum(selmat, axis=1, keepdims=True) > 0
        ctx_ref[0, 0, pl.ds(i * blk, blk), :] = jnp.where(issel, repl, csum)
        return jax.lax.slice(csum, (blk - 1, 0), (blk, dh))

    jax.lax.fori_loop(0, nblk, ctx_block, jnp.zeros((1, dh), f32))


def _out_body(ctx_ref, wo_ref, bo_ref, out_ref):
    h = pl.program_id(1)
    part = jax.lax.dot_general(ctx_ref[0, 0], wo_ref[0], (((1,), (0,)), ((), ())),
                               preferred_element_type=jnp.float32)

    @pl.when(h == 0)
    def _():
        out_ref[0] = part + bo_ref[0]

    @pl.when(h != 0)
    def _():
        out_ref[0] = out_ref[0] + part


def kernel(x, Wq, bq, Wk, bk, Wv, bv, Wo, bo, index_sample):
    B, L, D = x.shape
    H = N_HEADS
    dh = D // H
    u = min(FACTOR * int(math.ceil(math.log(L))), L)
    u_part = index_sample.shape[1]
    blk = 128
    f32 = jnp.float32

    idx = index_sample.astype(jnp.int32)
    wq_r = Wq.reshape(D, H, dh).transpose(1, 0, 2)
    wk_r = Wk.reshape(D, H, dh).transpose(1, 0, 2)
    wv_r = Wv.reshape(D, H, dh).transpose(1, 0, 2)
    wo_r = Wo.reshape(H, dh, D)
    bq_r = bq.reshape(H, 1, dh)
    bk_r = bk.reshape(H, 1, dh)
    bv_r = bv.reshape(H, 1, dh)
    bo_r = bo.reshape(1, 1, D)

    qkv_shape = jax.ShapeDtypeStruct((B, H, L, dh), f32)
    q, k, v = pl.pallas_call(
        _qkv_body,
        grid=(B, H),
        in_specs=[
            pl.BlockSpec((1, L, D), lambda b, h: (b, 0, 0)),
            pl.BlockSpec((1, D, dh), lambda b, h: (h, 0, 0)),
            pl.BlockSpec((1, D, dh), lambda b, h: (h, 0, 0)),
            pl.BlockSpec((1, D, dh), lambda b, h: (h, 0, 0)),
            pl.BlockSpec((1, 1, dh), lambda b, h: (h, 0, 0)),
            pl.BlockSpec((1, 1, dh), lambda b, h: (h, 0, 0)),
            pl.BlockSpec((1, 1, dh), lambda b, h: (h, 0, 0)),
        ],
        out_specs=[
            pl.BlockSpec((1, 1, L, dh), lambda b, h: (b, h, 0, 0)),
            pl.BlockSpec((1, 1, L, dh), lambda b, h: (b, h, 0, 0)),
            pl.BlockSpec((1, 1, L, dh), lambda b, h: (b, h, 0, 0)),
        ],
        out_shape=[qkv_shape, qkv_shape, qkv_shape],
    )(x, wq_r, wk_r, wv_r, bq_r, bk_r, bv_r)

    w = pl.pallas_call(
        _wcount_body,
        grid=(L // blk,),
        in_specs=[pl.BlockSpec((blk, u_part), lambda i: (i, 0))],
        out_specs=pl.BlockSpec((blk, L), lambda i: (i, 0)),
        out_shape=jax.ShapeDtypeStruct((L, L), jnp.int8),
    )(idx)

    ctx = pl.pallas_call(
        functools.partial(_attn_body, u=u, blk=blk),
        grid=(B * H,),
        in_specs=[
            pl.BlockSpec((1, 1, L, dh), lambda g: (g // H, g % H, 0, 0)),
            pl.BlockSpec((1, 1, L, dh), lambda g: (g // H, g % H, 0, 0)),
            pl.BlockSpec((1, 1, L, dh), lambda g: (g // H, g % H, 0, 0)),
            pl.BlockSpec((L, L), lambda g: (0, 0)),
        ],
        out_specs=pl.BlockSpec((1, 1, L, dh), lambda g: (g // H, g % H, 0, 0)),
        out_shape=jax.ShapeDtypeStruct((B, H, L, dh), f32),
        scratch_shapes=[pltpu.VMEM((L // blk, blk), f32)],
    )(q, k, v, w)

    out = pl.pallas_call(
        _out_body,
        grid=(B, H),
        in_specs=[
            pl.BlockSpec((1, 1, L, dh), lambda b, h: (b, h, 0, 0)),
            pl.BlockSpec((1, dh, D), lambda b, h: (h, 0, 0)),
            pl.BlockSpec((1, 1, D), lambda b, h: (0, 0, 0)),
        ],
        out_specs=pl.BlockSpec((1, L, D), lambda b, h: (b, 0, 0)),
        out_shape=jax.ShapeDtypeStruct((B, L, D), f32),
    )(ctx, wo_r, bo_r)
    return out


# speedup vs baseline: 4.5755x; 1.3547x over previous
import functools
import math

import jax
import jax.numpy as jnp
from jax import lax
from jax.experimental import pallas as pl
from jax.experimental.pallas import tpu as pltpu
from jax.experimental.pallas import tpu_sc as plsc

N_HEADS = 12
FACTOR = 5
NEG_INF = float("-inf")


def _qkv_body(x_ref, wq_ref, wk_ref, wv_ref, bq_ref, bk_ref, bv_ref,
              q_ref, k_ref, v_ref):
    xb = x_ref[0]
    cn = (((1,), (0,)), ((), ()))
    f32 = jnp.float32
    q_ref[0, 0] = jax.lax.dot_general(xb, wq_ref[0], cn,
                                      preferred_element_type=f32) + bq_ref[0]
    k_ref[0, 0] = jax.lax.dot_general(xb, wk_ref[0], cn,
                                      preferred_element_type=f32) + bk_ref[0]
    v_ref[0, 0] = jax.lax.dot_general(xb, wv_ref[0], cn,
                                      preferred_element_type=f32) + bv_ref[0]


def _make_wbuild_sc(L, u_part):
    info = plsc.get_sparse_core_info()
    nw = info.num_cores * info.num_subcores
    rows_w = L // nw
    half = 32
    nchunks = rows_w // half
    mesh = plsc.VectorSubcoreMesh(core_axis_name="c", subcore_axis_name="s")

    upad = ((u_part + 15) // 16) * 16
    ngrp = upad // 16
    rem = u_part - (ngrp - 1) * 16

    @functools.partial(
        pl.kernel, mesh=mesh,
        out_type=jax.ShapeDtypeStruct((L * L,), jnp.float32),
        compiler_params=pltpu.CompilerParams(needs_layout_passes=False),
        scratch_types=[
            pltpu.VMEM((half * upad,), jnp.int32),
            pltpu.VMEM((half * L,), jnp.float32),
        ],
    )
    def wbuild(idx_hbm, zero_hbm, w_hbm, idx_v, buf_v):
        wid = lax.axis_index("s") * info.num_cores + lax.axis_index("c")
        base = wid * rows_w
        lane = lax.iota(jnp.int32, 16)
        ones = jnp.ones((16,), jnp.float32)
        tailmask = lane < rem
        for c in range(nchunks):
            r0 = base + c * half
            pltpu.sync_copy(idx_hbm.at[pl.ds(r0 * upad, half * upad)], idx_v)
            pltpu.sync_copy(zero_hbm, buf_v)
            for r in range(half):
                rowbase = r * L
                for g in range(ngrp):
                    vals = idx_v[pl.ds(r * upad + g * 16, 16)]
                    if g < ngrp - 1 or rem == 16:
                        plsc.addupdate_scatter(buf_v, [rowbase + vals], ones)
                    else:
                        plsc.addupdate_scatter(buf_v, [rowbase + vals], ones,
                                               mask=tailmask)
            pltpu.sync_copy(buf_v, w_hbm.at[pl.ds(r0 * L, half * L)])

    return wbuild


def _wcount_body(idx_ref, w_ref):
    blk, u_part = idx_ref.shape
    l_keys = w_ref.shape[1]
    idx_f = idx_ref[...].astype(jnp.float32)
    k_iota = jax.lax.broadcasted_iota(jnp.int32, (blk, l_keys), 1).astype(jnp.float32)
    s_iota = jax.lax.broadcasted_iota(jnp.int32, (u_part, 1), 0)

    def body(s, acc):
        e = (s_iota == s).astype(jnp.float32)
        col = jax.lax.dot_general(idx_f, e, (((1,), (0,)), ((), ())),
                                  precision=jax.lax.Precision.HIGHEST,
                                  preferred_element_type=jnp.float32)
        return acc + (k_iota == col).astype(jnp.float32)

    acc = jax.lax.fori_loop(0, u_part, body, jnp.zeros((blk, l_keys), jnp.float32))
    w_ref[...] = acc.astype(jnp.int8)


def _attn_body(q_ref, k_ref, v_ref, w_ref, ctx_ref, m_ref, *, u, blk):
    L, dh = q_ref.shape[2], q_ref.shape[3]
    nblk = L // blk
    upad = 64
    cn_t = (((1,), (1,)), ((), ()))
    cn_m = (((1,), (0,)), ((), ()))
    f32 = jnp.float32

    kv = k_ref[0, 0]
    vv = v_ref[0, 0]

    def m_block(i, _):
        qb = q_ref[0, 0, pl.ds(i * blk, blk), :]
        s = jax.lax.dot_general(qb, kv, cn_t, preferred_element_type=f32)
        cnt = w_ref[pl.ds(i * blk, blk), :].astype(f32)
        smax = jnp.max(jnp.where(cnt > 0, s, NEG_INF), axis=1)
        ssum = jnp.sum(s * cnt, axis=1)
        mb = smax - ssum * (1.0 / L)
        m_ref[pl.ds(i, 1), :] = mb.reshape(1, blk)
        return 0

    jax.lax.fori_loop(0, nblk, m_block, 0)
    m_all = m_ref[...]

    pos = (jax.lax.broadcasted_iota(jnp.int32, (nblk, blk), 0) * blk
           + jax.lax.broadcasted_iota(jnp.int32, (nblk, blk), 1))
    lane = jax.lax.broadcasted_iota(jnp.int32, (1, upad), 1)

    def topk_body(j, carry):
        m_cur, tops = carry
        mmax = jnp.max(m_cur)
        sel = jnp.min(jnp.where(m_cur == mmax, pos, 2 * L))
        tops = jnp.where(lane == j, sel, tops)
        m_cur = jnp.where(pos == sel, NEG_INF, m_cur)
        return m_cur, tops

    tops0 = jnp.full((1, upad), 2 * L, jnp.int32)
    _, tops = jax.lax.fori_loop(0, u, topk_body, (m_all, tops0))
    tops_col = tops.reshape(upad, 1)

    k_iota = jax.lax.broadcasted_iota(jnp.int32, (upad, L), 1)
    onehot = (k_iota == tops_col).astype(f32)
    q_sel = jax.lax.dot_general(onehot, q_ref[0, 0], cn_m,
                                precision=jax.lax.Precision.HIGHEST, preferred_element_type=f32)
    scores = jax.lax.dot_general(q_sel, kv, cn_t, preferred_element_type=f32)
    scores = scores * (1.0 / math.sqrt(dh))
    scores = jnp.where(k_iota > tops_col, NEG_INF, scores)
    scores = scores - jnp.max(scores, axis=1, keepdims=True)
    p = jnp.exp(scores)
    attn = p / jnp.sum(p, axis=1, keepdims=True)
    update = jax.lax.dot_general(attn, vv, cn_m,
                                 preferred_element_type=f32)

    tri = (jax.lax.broadcasted_iota(jnp.int32, (blk, blk), 0)
           >= jax.lax.broadcasted_iota(jnp.int32, (blk, blk), 1)).astype(f32)
    row_iota = jax.lax.broadcasted_iota(jnp.int32, (blk, 1), 0)

    def ctx_block(i, carry):
        vb = v_ref[0, 0, pl.ds(i * blk, blk), :]
        csum = jax.lax.dot_general(tri, vb, cn_m,
                                   precision=jax.lax.Precision.HIGHEST, preferred_element_type=f32) + carry
        offs = row_iota + i * blk
        selmat = (offs == tops).astype(f32)
        repl = jax.lax.dot_general(selmat, update, cn_m,
                                   precision=jax.lax.Precision.HIGHEST, preferred_element_type=f32)
        issel = jnp.sum(selmat, axis=1, keepdims=True) > 0
        ctx_ref[0, 0, pl.ds(i * blk, blk), :] = jnp.where(issel, repl, csum)
        return jax.lax.slice(csum, (blk - 1, 0), (blk, dh))

    jax.lax.fori_loop(0, nblk, ctx_block, jnp.zeros((1, dh), f32))


def _out_body(ctx_ref, wo_ref, bo_ref, out_ref):
    h = pl.program_id(1)
    part = jax.lax.dot_general(ctx_ref[0, 0], wo_ref[0], (((1,), (0,)), ((), ())),
                               preferred_element_type=jnp.float32)

    @pl.when(h == 0)
    def _():
        out_ref[0] = part + bo_ref[0]

    @pl.when(h != 0)
    def _():
        out_ref[0] = out_ref[0] + part


def kernel(x, Wq, bq, Wk, bk, Wv, bv, Wo, bo, index_sample):
    B, L, D = x.shape
    H = N_HEADS
    dh = D // H
    u = min(FACTOR * int(math.ceil(math.log(L))), L)
    u_part = index_sample.shape[1]
    blk = 128
    f32 = jnp.float32

    idx = index_sample.astype(jnp.int32)
    wq_r = Wq.reshape(D, H, dh).transpose(1, 0, 2)
    wk_r = Wk.reshape(D, H, dh).transpose(1, 0, 2)
    wv_r = Wv.reshape(D, H, dh).transpose(1, 0, 2)
    wo_r = Wo.reshape(H, dh, D)
    bq_r = bq.reshape(H, 1, dh)
    bk_r = bk.reshape(H, 1, dh)
    bv_r = bv.reshape(H, 1, dh)
    bo_r = bo.reshape(1, 1, D)

    qkv_shape = jax.ShapeDtypeStruct((B, H, L, dh), f32)
    q, k, v = pl.pallas_call(
        _qkv_body,
        grid=(B, H),
        in_specs=[
            pl.BlockSpec((1, L, D), lambda b, h: (b, 0, 0)),
            pl.BlockSpec((1, D, dh), lambda b, h: (h, 0, 0)),
            pl.BlockSpec((1, D, dh), lambda b, h: (h, 0, 0)),
            pl.BlockSpec((1, D, dh), lambda b, h: (h, 0, 0)),
            pl.BlockSpec((1, 1, dh), lambda b, h: (h, 0, 0)),
            pl.BlockSpec((1, 1, dh), lambda b, h: (h, 0, 0)),
            pl.BlockSpec((1, 1, dh), lambda b, h: (h, 0, 0)),
        ],
        out_specs=[
            pl.BlockSpec((1, 1, L, dh), lambda b, h: (b, h, 0, 0)),
            pl.BlockSpec((1, 1, L, dh), lambda b, h: (b, h, 0, 0)),
            pl.BlockSpec((1, 1, L, dh), lambda b, h: (b, h, 0, 0)),
        ],
        out_shape=[qkv_shape, qkv_shape, qkv_shape],
    )(x, wq_r, wk_r, wv_r, bq_r, bk_r, bv_r)

    upad_s = ((u_part + 15) // 16) * 16
    idx_pad = jnp.pad(idx, ((0, 0), (0, upad_s - u_part)))
    w = _make_wbuild_sc(L, u_part)(
        idx_pad.reshape(L * upad_s), jnp.zeros((32 * L,), f32)).reshape(L, L)

    ctx = pl.pallas_call(
        functools.partial(_attn_body, u=u, blk=blk),
        grid=(B * H,),
        in_specs=[
            pl.BlockSpec((1, 1, L, dh), lambda g: (g // H, g % H, 0, 0)),
            pl.BlockSpec((1, 1, L, dh), lambda g: (g // H, g % H, 0, 0)),
            pl.BlockSpec((1, 1, L, dh), lambda g: (g // H, g % H, 0, 0)),
            pl.BlockSpec((L, L), lambda g: (0, 0)),
        ],
        out_specs=pl.BlockSpec((1, 1, L, dh), lambda g: (g // H, g % H, 0, 0)),
        out_shape=jax.ShapeDtypeStruct((B, H, L, dh), f32),
        scratch_shapes=[pltpu.VMEM((L // blk, blk), f32)],
    )(q, k, v, w)

    out = pl.pallas_call(
        _out_body,
        grid=(B, H),
        in_specs=[
            pl.BlockSpec((1, 1, L, dh), lambda b, h: (b, h, 0, 0)),
            pl.BlockSpec((1, dh, D), lambda b, h: (h, 0, 0)),
            pl.BlockSpec((1, 1, D), lambda b, h: (0, 0, 0)),
        ],
        out_specs=pl.BlockSpec((1, L, D), lambda b, h: (b, 0, 0)),
        out_shape=jax.ShapeDtypeStruct((B, L, D), f32),
    )(ctx, wo_r, bo_r)
    return out


# speedup vs baseline: 6.1356x; 1.3410x over previous
import functools
import math

import jax
import jax.numpy as jnp
from jax import lax
from jax.experimental import pallas as pl
from jax.experimental.pallas import tpu as pltpu
from jax.experimental.pallas import tpu_sc as plsc

N_HEADS = 12
FACTOR = 5
NEG_INF = float("-inf")


def _qkv_body(x_ref, wq_ref, wk_ref, wv_ref, bq_ref, bk_ref, bv_ref,
              q_ref, k_ref, v_ref):
    xb = x_ref[0]
    cn = (((1,), (0,)), ((), ()))
    f32 = jnp.float32
    q_ref[0, 0] = jax.lax.dot_general(xb, wq_ref[0], cn,
                                      preferred_element_type=f32) + bq_ref[0]
    k_ref[0, 0] = jax.lax.dot_general(xb, wk_ref[0], cn,
                                      preferred_element_type=f32) + bk_ref[0]
    v_ref[0, 0] = jax.lax.dot_general(xb, wv_ref[0], cn,
                                      preferred_element_type=f32) + bv_ref[0]


def _make_wbuild_sc(L, u_part):
    info = plsc.get_sparse_core_info()
    nw = info.num_cores * info.num_subcores
    rows_w = L // nw
    half = 32
    nchunks = rows_w // half
    mesh = plsc.VectorSubcoreMesh(core_axis_name="c", subcore_axis_name="s")

    upad = ((u_part + 15) // 16) * 16
    ngrp = upad // 16
    rem = u_part - (ngrp - 1) * 16

    @functools.partial(
        pl.kernel, mesh=mesh,
        out_type=jax.ShapeDtypeStruct((L * L,), jnp.float32),
        compiler_params=pltpu.CompilerParams(needs_layout_passes=False),
        scratch_types=[
            pltpu.VMEM((half * upad,), jnp.int32),
            pltpu.VMEM((half * L,), jnp.float32),
        ],
    )
    def wbuild(idx_hbm, zero_hbm, w_hbm, idx_v, buf_v):
        wid = lax.axis_index("s") * info.num_cores + lax.axis_index("c")
        base = wid * rows_w
        lane = lax.iota(jnp.int32, 16)
        ones = jnp.ones((16,), jnp.float32)
        tailmask = lane < rem
        for c in range(nchunks):
            r0 = base + c * half
            pltpu.sync_copy(idx_hbm.at[pl.ds(r0 * upad, half * upad)], idx_v)
            pltpu.sync_copy(zero_hbm, buf_v)
            for r in range(half):
                rowbase = r * L
                for g in range(ngrp):
                    vals = idx_v[pl.ds(r * upad + g * 16, 16)]
                    if g < ngrp - 1 or rem == 16:
                        plsc.addupdate_scatter(buf_v, [rowbase + vals], ones)
                    else:
                        plsc.addupdate_scatter(buf_v, [rowbase + vals], ones,
                                               mask=tailmask)
            pltpu.sync_copy(buf_v, w_hbm.at[pl.ds(r0 * L, half * L)])

    return wbuild


def _wcount_body(idx_ref, w_ref):
    blk, u_part = idx_ref.shape
    l_keys = w_ref.shape[1]
    idx_f = idx_ref[...].astype(jnp.float32)
    k_iota = jax.lax.broadcasted_iota(jnp.int32, (blk, l_keys), 1).astype(jnp.float32)
    s_iota = jax.lax.broadcasted_iota(jnp.int32, (u_part, 1), 0)

    def body(s, acc):
        e = (s_iota == s).astype(jnp.float32)
        col = jax.lax.dot_general(idx_f, e, (((1,), (0,)), ((), ())),
                                  precision=jax.lax.Precision.HIGHEST,
                                  preferred_element_type=jnp.float32)
        return acc + (k_iota == col).astype(jnp.float32)

    acc = jax.lax.fori_loop(0, u_part, body, jnp.zeros((blk, l_keys), jnp.float32))
    w_ref[...] = acc.astype(jnp.int8)


def _m_body(q_ref, k_ref, w_ref, m_ref, *, blk):
    L = q_ref.shape[2]
    nblk = L // blk
    cn_t = (((1,), (1,)), ((), ()))
    f32 = jnp.float32
    kv = k_ref[0, 0]

    def m_block(i, _):
        qb = q_ref[0, 0, pl.ds(i * blk, blk), :]
        s = jax.lax.dot_general(qb, kv, cn_t, preferred_element_type=f32)
        cnt = w_ref[pl.ds(i * blk, blk), :]
        smax = jnp.max(jnp.where(cnt > 0, s, NEG_INF), axis=1)
        ssum = jnp.sum(s * cnt, axis=1)
        mb = smax - ssum * (1.0 / L)
        m_ref[0, :, pl.ds(i * blk, blk)] = mb.reshape(1, blk)
        return 0

    jax.lax.fori_loop(0, nblk, m_block, 0)


def _topk_body(m_ref, t_ref, *, u, upad):
    G, L = m_ref.shape[0], m_ref.shape[2]
    f32 = jnp.float32
    m = m_ref[...].reshape(G, L)
    pos = jax.lax.broadcasted_iota(jnp.int32, (G, L), 1)
    lane = jax.lax.broadcasted_iota(jnp.int32, (G, upad), 1)

    def topk_body(j, carry):
        m_cur, tops = carry
        rmax = jnp.max(m_cur, axis=1, keepdims=True)
        sel = jnp.min(jnp.where(m_cur == rmax, pos, 2 * L), axis=1,
                      keepdims=True)
        tops = jnp.where(lane == j, sel, tops)
        m_cur = jnp.where(pos == sel, NEG_INF, m_cur)
        return m_cur, tops

    tops0 = jnp.full((G, upad), 2 * L, jnp.int32)
    _, tops = jax.lax.fori_loop(0, u, topk_body, (m, tops0))
    t_ref[...] = tops.reshape(G, 1, upad)


def _finish_body(q_ref, k_ref, v_ref, t_ref, wo_ref, bo_ref, out_ref, *, blk):
    L, dh = q_ref.shape[2], q_ref.shape[3]
    D = out_ref.shape[2]
    nblk = L // blk
    upad = t_ref.shape[2]
    cn_t = (((1,), (1,)), ((), ()))
    cn_m = (((1,), (0,)), ((), ()))
    hp = jax.lax.Precision.HIGHEST
    f32 = jnp.float32
    h = pl.program_id(1)

    kv = k_ref[0, 0]
    vv = v_ref[0, 0]
    wo = wo_ref[0]
    tops = t_ref[0]
    tops_col = tops.reshape(upad, 1)

    k_iota = jax.lax.broadcasted_iota(jnp.int32, (upad, L), 1)
    onehot = (k_iota == tops_col).astype(f32)
    q_sel = jax.lax.dot_general(onehot, q_ref[0, 0], cn_m, precision=hp,
                                preferred_element_type=f32)
    scores = jax.lax.dot_general(q_sel, kv, cn_t, preferred_element_type=f32)
    scores = scores * (1.0 / math.sqrt(dh))
    scores = jnp.where(k_iota > tops_col, NEG_INF, scores)
    scores = scores - jnp.max(scores, axis=1, keepdims=True)
    p = jnp.exp(scores)
    attn = p / jnp.sum(p, axis=1, keepdims=True)
    update = jax.lax.dot_general(attn, vv, cn_m,
                                 preferred_element_type=f32)

    tri = (jax.lax.broadcasted_iota(jnp.int32, (blk, blk), 0)
           >= jax.lax.broadcasted_iota(jnp.int32, (blk, blk), 1)).astype(f32)
    row_iota = jax.lax.broadcasted_iota(jnp.int32, (blk, 1), 0)

    def ctx_block(i, carry):
        vb = v_ref[0, 0, pl.ds(i * blk, blk), :]
        csum = jax.lax.dot_general(tri, vb, cn_m, precision=hp,
                                   preferred_element_type=f32) + carry
        offs = row_iota + i * blk
        selmat = (offs == tops).astype(f32)
        repl = jax.lax.dot_general(selmat, update, cn_m, precision=hp,
                                   preferred_element_type=f32)
        issel = jnp.sum(selmat, axis=1, keepdims=True) > 0
        ctx_blk = jnp.where(issel, repl, csum)
        part = jax.lax.dot_general(ctx_blk, wo, cn_m,
                                   preferred_element_type=f32)

        @pl.when(h == 0)
        def _():
            out_ref[0, pl.ds(i * blk, blk), :] = part + bo_ref[0]

        @pl.when(h != 0)
        def _():
            out_ref[0, pl.ds(i * blk, blk), :] = (
                out_ref[0, pl.ds(i * blk, blk), :] + part)

        return jax.lax.slice(csum, (blk - 1, 0), (blk, dh))

    jax.lax.fori_loop(0, nblk, ctx_block, jnp.zeros((1, dh), f32))


def kernel(x, Wq, bq, Wk, bk, Wv, bv, Wo, bo, index_sample):
    B, L, D = x.shape
    H = N_HEADS
    dh = D // H
    u = min(FACTOR * int(math.ceil(math.log(L))), L)
    u_part = index_sample.shape[1]
    blk = 128
    f32 = jnp.float32

    idx = index_sample.astype(jnp.int32)
    wq_r = Wq.reshape(D, H, dh).transpose(1, 0, 2)
    wk_r = Wk.reshape(D, H, dh).transpose(1, 0, 2)
    wv_r = Wv.reshape(D, H, dh).transpose(1, 0, 2)
    wo_r = Wo.reshape(H, dh, D)
    bq_r = bq.reshape(H, 1, dh)
    bk_r = bk.reshape(H, 1, dh)
    bv_r = bv.reshape(H, 1, dh)
    bo_r = bo.reshape(1, 1, D)

    qkv_shape = jax.ShapeDtypeStruct((B, H, L, dh), f32)
    q, k, v = pl.pallas_call(
        _qkv_body,
        grid=(B, H),
        in_specs=[
            pl.BlockSpec((1, L, D), lambda b, h: (b, 0, 0)),
            pl.BlockSpec((1, D, dh), lambda b, h: (h, 0, 0)),
            pl.BlockSpec((1, D, dh), lambda b, h: (h, 0, 0)),
            pl.BlockSpec((1, D, dh), lambda b, h: (h, 0, 0)),
            pl.BlockSpec((1, 1, dh), lambda b, h: (h, 0, 0)),
            pl.BlockSpec((1, 1, dh), lambda b, h: (h, 0, 0)),
            pl.BlockSpec((1, 1, dh), lambda b, h: (h, 0, 0)),
        ],
        out_specs=[
            pl.BlockSpec((1, 1, L, dh), lambda b, h: (b, h, 0, 0)),
            pl.BlockSpec((1, 1, L, dh), lambda b, h: (b, h, 0, 0)),
            pl.BlockSpec((1, 1, L, dh), lambda b, h: (b, h, 0, 0)),
        ],
        out_shape=[qkv_shape, qkv_shape, qkv_shape],
    )(x, wq_r, wk_r, wv_r, bq_r, bk_r, bv_r)

    upad_s = ((u_part + 15) // 16) * 16
    idx_pad = jnp.pad(idx, ((0, 0), (0, upad_s - u_part)))
    w = _make_wbuild_sc(L, u_part)(
        idx_pad.reshape(L * upad_s), jnp.zeros((32 * L,), f32)).reshape(L, L)

    m = pl.pallas_call(
        functools.partial(_m_body, blk=blk),
        grid=(B * H,),
        in_specs=[
            pl.BlockSpec((1, 1, L, dh), lambda g: (g // H, g % H, 0, 0)),
            pl.BlockSpec((1, 1, L, dh), lambda g: (g // H, g % H, 0, 0)),
            pl.BlockSpec((L, L), lambda g: (0, 0)),
        ],
        out_specs=pl.BlockSpec((1, 1, L), lambda g: (g, 0, 0)),
        out_shape=jax.ShapeDtypeStruct((B * H, 1, L), f32),
    )(q, k, w)

    upad = 64
    tops = pl.pallas_call(
        functools.partial(_topk_body, u=u, upad=upad),
        in_specs=[pl.BlockSpec((B * H, 1, L), lambda: (0, 0, 0))],
        out_specs=pl.BlockSpec((B * H, 1, upad), lambda: (0, 0, 0)),
        out_shape=jax.ShapeDtypeStruct((B * H, 1, upad), jnp.int32),
    )(m)

    out = pl.pallas_call(
        functools.partial(_finish_body, blk=blk),
        grid=(B, H),
        in_specs=[
            pl.BlockSpec((1, 1, L, dh), lambda b, h: (b, h, 0, 0)),
            pl.BlockSpec((1, 1, L, dh), lambda b, h: (b, h, 0, 0)),
            pl.BlockSpec((1, 1, L, dh), lambda b, h: (b, h, 0, 0)),
            pl.BlockSpec((1, 1, upad), lambda b, h: (b * H + h, 0, 0)),
            pl.BlockSpec((1, dh, D), lambda b, h: (h, 0, 0)),
            pl.BlockSpec((1, 1, D), lambda b, h: (0, 0, 0)),
        ],
        out_specs=pl.BlockSpec((1, L, D), lambda b, h: (b, 0, 0)),
        out_shape=jax.ShapeDtypeStruct((B, L, D), f32),
    )(q, k, v, tops, wo_r, bo_r)
    return out
